# baseline (speedup 1.0000x reference)
import jax
import jax.numpy as jnp
from jax import lax
from jax.experimental import pallas as pl
from jax.experimental.pallas import tpu as pltpu
from jax.experimental.pallas import tpu_sc as plsc

N_NODES = 10000
D = 128
E = 320000
K = 32
NC = 2
NS = 16
NW = NC * NS
EPT = 10000
NCH = 320
PAD = NCH * K - EPT
BLK = 16
NBL = NCH // BLK
NB = 8
DROWS = 64
ROWS_ACC = N_NODES + DROWS
NCT = 10
CW = N_NODES // NCT
BM = 1000


def _make_sc_agg(with_deg: bool):
    mesh = plsc.VectorSubcoreMesh(core_axis_name="c", subcore_axis_name="s")
    agg_t = jax.ShapeDtypeStruct((NC * N_NODES, D), jnp.float32)
    out_type = ([agg_t, jax.ShapeDtypeStruct((NC * N_NODES,), jnp.float32)]
                if with_deg else agg_t)
    scratch = (
        [pltpu.VMEM_SHARED((ROWS_ACC, D), jnp.float32)]
        + [pltpu.VMEM((BLK, 2 * K), jnp.int32)] * 2
        + [pltpu.VMEM((K, D), jnp.float32)] * NB
        + [pltpu.SemaphoreType.DMA] * (2 * NB + 2)
    )
    if with_deg:
        scratch += [
            pltpu.VMEM_SHARED((ROWS_ACC,), jnp.float32),
            pltpu.VMEM((K,), jnp.float32),
            pltpu.VMEM((CW,), jnp.float32),
        ]

    def body(y_hbm, z128_hbm, z1_hbm, ed_hbm, agg_out, *rest):
        if with_deg:
            deg_out = rest[0]
            rest = rest[1:]
        acc_sh = rest[0]
        ed = rest[1:3]
        rows = rest[3:3 + NB]
        semg = rest[3 + NB:3 + 2 * NB]
        sems = rest[3 + 2 * NB:3 + 3 * NB]
        seme = rest[3 + 3 * NB:3 + 3 * NB + 2]
        if with_deg:
            deg_sh, ones_v, bounce_v = rest[3 + 3 * NB + 2:]
        c = lax.axis_index("c")
        s = lax.axis_index("s")
        w = c * NS + s

        if with_deg:
            ov = jnp.ones((16,), jnp.float32)
            for j in range(K // 16):
                ones_v[pl.ds(j * 16, 16)] = ov

        @pl.when(s < NCT)
        def _zero():
            pltpu.sync_copy(z128_hbm, acc_sh.at[pl.ds(s * CW, CW)])
            if with_deg:
                pltpu.sync_copy(z1_hbm, bounce_v)
                pltpu.sync_copy(bounce_v, deg_sh.at[pl.ds(s * CW, CW)])

        plsc.subcore_barrier()

        def stage(blk_idx, e):
            return pltpu.async_copy(
                ed_hbm.at[pl.ds(w * NCH + blk_idx * BLK, BLK)], ed[e],
                seme[e])

        def stage_wait(blk_idx, e):
            pltpu.make_async_copy(
                ed_hbm.at[pl.ds(w * NCH + blk_idx * BLK, BLK)], ed[e],
                seme[e]).wait()

        def gather(si, e):
            pltpu.async_copy(y_hbm.at[ed[e].at[si % BLK, pl.ds(K, K)]],
                             rows[si % NB], semg[si % NB])

        def gather_wait(si, e):
            pltpu.make_async_copy(y_hbm.at[ed[e].at[si % BLK, pl.ds(K, K)]],
                                  rows[si % NB], semg[si % NB]).wait()

        def scatter(si, e):
            didx = ed[e].at[si % BLK, pl.ds(0, K)]
            b = si % NB
            pltpu.async_copy(rows[b], acc_sh.at[didx], sems[b], add=True)
            if with_deg:
                pltpu.async_copy(ones_v, deg_sh.at[didx], sems[b], add=True)

        def scatter_wait(si, e):
            didx = ed[e].at[si % BLK, pl.ds(0, K)]
            b = si % NB
            pltpu.make_async_copy(rows[b], acc_sh.at[didx], sems[b]).wait()
            if with_deg:
                pltpu.make_async_copy(ones_v, deg_sh.at[didx],
                                      sems[b]).wait()

        stage(0, 0)
        stage_wait(0, 0)
        for i in range(NB - 2):
            gather(i, 0)

        def pair_body(jj, carry):
            g0 = 2 * BLK * jj
            for i in range(2 * BLK):
                g = g0 + i
                e = (i // BLK) % 2
                if i == 2:
                    stage(2 * jj + 1, 1)
                if i == BLK + 2:
                    @pl.when(g + 2 * BLK - 2 < NCH)
                    def _stg0():
                        stage(2 * jj + 2, 0)
                if i == BLK - NB + 2:
                    stage_wait(2 * jj + 1, 1)
                if i == 2 * BLK - NB + 2:
                    @pl.when(g + NB - 2 + BLK - ((NB - 2 + BLK) % BLK)
                             < NCH)
                    def _stw0():
                        stage_wait(2 * jj + 2, 0)
                if i < 2:
                    @pl.when(g >= 2)
                    def _drain():
                        scatter_wait(i - 2, ((i - 2) // BLK) % 2)
                else:
                    scatter_wait(i - 2, ((i - 2) // BLK) % 2)
                fe = ((i + NB - 2) // BLK) % 2
                if i >= 2 * BLK - (NB - 2):
                    @pl.when(g + NB - 2 < NCH)
                    def _ahead():
                        gather(i + NB - 2, fe)
                else:
                    gather(i + NB - 2, fe)
                gather_wait(i, e)
                scatter(i, e)
            return carry

        lax.fori_loop(0, NBL // 2, pair_body, 0)
        scatter_wait(2 * BLK - 2, 1)
        scatter_wait(2 * BLK - 1, 1)
        plsc.subcore_barrier()

        @pl.when(s < NCT)
        def _copy_out():
            base = c * N_NODES + s * CW
            pltpu.sync_copy(acc_sh.at[pl.ds(s * CW, CW)],
                            agg_out.at[pl.ds(base, CW)])
            if with_deg:
                pltpu.sync_copy(deg_sh.at[pl.ds(s * CW, CW)], bounce_v)
                pltpu.sync_copy(bounce_v, deg_out.at[pl.ds(base, CW)])

    return pl.kernel(body, out_type=out_type, mesh=mesh,
                     scratch_types=scratch)


_sc_agg_deg = _make_sc_agg(True)
_sc_agg = _make_sc_agg(False)


def _tc_pre_body(x_ref, wl_ref, wr_ref, b_ref, y_ref, r_ref):
    x = x_ref[...]
    y_ref[...] = jnp.dot(x, wl_ref[...], preferred_element_type=jnp.float32)
    r_ref[...] = (jnp.dot(x, wr_ref[...], preferred_element_type=jnp.float32)
                  + b_ref[...])


def _tc_pre(x, wl, wr, b):
    return pl.pallas_call(
        _tc_pre_body,
        grid=(N_NODES // BM,),
        in_specs=[pl.BlockSpec((BM, D), lambda i: (i, 0)),
                  pl.BlockSpec((D, D), lambda i: (0, 0)),
                  pl.BlockSpec((D, D), lambda i: (0, 0)),
                  pl.BlockSpec((1, D), lambda i: (0, 0))],
        out_specs=[pl.BlockSpec((BM, D), lambda i: (i, 0))] * 2,
        out_shape=[jax.ShapeDtypeStruct((N_NODES, D), jnp.float32)] * 2,
    )(x, wl, wr, b)


def _deg_col(d0_ref, d1_ref):
    d = d0_ref[0, 0, :] + d1_ref[0, 0, :]
    return jnp.maximum(d, 1.0).reshape(BM, 1)


def _tc_mid_body(a0_ref, a1_ref, d0_ref, d1_ref, r1_ref, wl_ref, wr_ref,
                 b_ref, y_ref, r_ref):
    agg = a0_ref[...] + a1_ref[...]
    h = jnp.maximum(agg / _deg_col(d0_ref, d1_ref) + r1_ref[...], 0.0)
    y_ref[...] = jnp.dot(h, wl_ref[...], preferred_element_type=jnp.float32)
    r_ref[...] = (jnp.dot(h, wr_ref[...], preferred_element_type=jnp.float32)
                  + b_ref[...])


def _tc_mid(agg, deg3, r1, wl, wr, b):
    nb = N_NODES // BM
    return pl.pallas_call(
        _tc_mid_body,
        grid=(nb,),
        in_specs=[pl.BlockSpec((BM, D), lambda i: (i, 0)),
                  pl.BlockSpec((BM, D), lambda i: (i + nb, 0)),
                  pl.BlockSpec((1, 1, BM), lambda i: (i, 0, 0)),
                  pl.BlockSpec((1, 1, BM), lambda i: (i + nb, 0, 0)),
                  pl.BlockSpec((BM, D), lambda i: (i, 0)),
                  pl.BlockSpec((D, D), lambda i: (0, 0)),
                  pl.BlockSpec((D, D), lambda i: (0, 0)),
                  pl.BlockSpec((1, D), lambda i: (0, 0))],
        out_specs=[pl.BlockSpec((BM, D), lambda i: (i, 0))] * 2,
        out_shape=[jax.ShapeDtypeStruct((N_NODES, D), jnp.float32)] * 2,
    )(agg, agg, deg3, deg3, r1, wl, wr, b)


def _tc_fin_body(a0_ref, a1_ref, d0_ref, d1_ref, r2_ref, o_ref):
    agg = a0_ref[...] + a1_ref[...]
    o_ref[...] = agg / _deg_col(d0_ref, d1_ref) + r2_ref[...]


def _tc_fin(agg, deg3, r2):
    nb = N_NODES // BM
    return pl.pallas_call(
        _tc_fin_body,
        grid=(nb,),
        in_specs=[pl.BlockSpec((BM, D), lambda i: (i, 0)),
                  pl.BlockSpec((BM, D), lambda i: (i + nb, 0)),
                  pl.BlockSpec((1, 1, BM), lambda i: (i, 0, 0)),
                  pl.BlockSpec((1, 1, BM), lambda i: (i + nb, 0, 0)),
                  pl.BlockSpec((BM, D), lambda i: (i, 0))],
        out_specs=pl.BlockSpec((BM, D), lambda i: (i, 0)),
        out_shape=jax.ShapeDtypeStruct((N_NODES, D), jnp.float32),
    )(agg, agg, deg3, deg3, r2)


def _pad_edges(idx, pad_vals):
    per_tile = idx.reshape(NW, EPT)
    padded = jnp.concatenate([per_tile, pad_vals], axis=1)
    return padded.reshape(NW, NCH, K)


def kernel(x, edge_index, W1_l, b1_l, W1_r, b1_r, W2_l, b2_l, W2_r, b2_r):
    src = edge_index[0].astype(jnp.int32)
    dst = edge_index[1].astype(jnp.int32)
    lane = jnp.arange(PAD, dtype=jnp.int32)[None, :]
    tile = jnp.arange(NW, dtype=jnp.int32)[:, None]
    src_pad = (tile * PAD + lane) % N_NODES
    dst_pad = N_NODES + (tile + lane) % DROWS
    src3 = _pad_edges(src, src_pad)
    dst3 = _pad_edges(dst, dst_pad)
    ed = jnp.concatenate([dst3, src3], axis=-1).reshape(NW * NCH, 2 * K)
    b1 = (b1_l + b1_r).reshape(1, D)
    b2 = (b2_l + b2_r).reshape(1, D)
    z128 = jnp.zeros((CW, D), jnp.float32)
    z1 = jnp.zeros((CW,), jnp.float32)

    y1, r1 = _tc_pre(x, W1_l, W1_r, b1)
    agg1, deg = _sc_agg_deg(y1, z128, z1, ed)
    deg3 = deg.reshape(NC * (N_NODES // BM), 1, BM)
    y2, r2 = _tc_mid(agg1, deg3, r1, W2_l, W2_r, b2)
    agg2 = _sc_agg(y2, z128, z1, ed)
    return _tc_fin(agg2, deg3, r2)
